# baseline (speedup 1.0000x reference)
import functools

import jax
import jax.numpy as jnp
from jax import lax
from jax.experimental import pallas as pl
from jax.experimental.pallas import tpu as pltpu
from jax.experimental.pallas import tpu_sc as plsc

SEQ = 1024
BATCH = 256
HALF = SEQ // 2
SENT = HALF + 1
NNODE = SENT + 1
EMB = 128
NLANE = 16
NSCAN = BATCH // NLANE
NW = 32
ROWS_PER_W = BATCH * SEQ // NW
CH = 256
NCH = ROWS_PER_W // CH


def _widx():
    return lax.axis_index("s") * 2 + lax.axis_index("c")


def _splat(v):
    return jnp.full((NLANE,), v, jnp.int32)


def _scan_body(sol_hbm, zv_hbm, zl_hbm, visit_hbm, t1_hbm, t2_hbm,
               sol_v, visit_v, t1_v, t2_v, nx_v, pv_v, il_v):
    wid = _widx()

    @pl.when(wid < NSCAN)
    def _():
        b0 = wid * NLANE
        pltpu.sync_copy(sol_hbm.at[pl.ds(b0 * SEQ, NLANE * SEQ)], sol_v)
        pltpu.sync_copy(zv_hbm, visit_v)
        pltpu.sync_copy(zv_hbm, t1_v)
        pltpu.sync_copy(zv_hbm, t2_v)
        pltpu.sync_copy(zl_hbm, il_v)
        nx_v[pl.ds(SENT * NLANE, NLANE)] = _splat(0)
        pv_v[pl.ds(0, NLANE)] = _splat(SENT)

        lane = lax.iota(jnp.int32, NLANE)
        lane_s = lane * SEQ
        sent = _splat(SENT)
        one = _splat(1)
        zero = _splat(0)

        def step(i, pre):
            cn = plsc.load_gather(sol_v, [lane_s + pre])
            vm = jnp.broadcast_to((i + 1) & (SEQ - 1), (NLANE,)).astype(jnp.int32)
            plsc.store_scatter(visit_v, [lane_s + cn], vm)
            push = (cn >= 1) & (cn <= HALF)
            pop = cn > HALF
            act = push | pop
            r = jnp.where(push, cn, cn - HALF)
            ra = r * NLANE + lane
            il = plsc.load_gather(il_v, [ra])
            rem = act & (il > 0)
            rn = plsc.load_gather(nx_v, [ra])
            rp = plsc.load_gather(pv_v, [ra])
            plsc.store_scatter(nx_v, [rp * NLANE + lane], rn, mask=rem)
            plsc.store_scatter(pv_v, [rn * NLANE + lane], rp, mask=rem)
            sa = SENT * NLANE + lane
            f = plsc.load_gather(nx_v, [sa])
            g = plsc.load_gather(nx_v, [f * NLANE + lane])
            ca = cn * NLANE + lane
            plsc.store_scatter(nx_v, [sa], cn, mask=push)
            plsc.store_scatter(pv_v, [ca], sent, mask=push)
            plsc.store_scatter(nx_v, [ca], f, mask=push)
            plsc.store_scatter(pv_v, [f * NLANE + lane], cn, mask=push)
            plsc.store_scatter(il_v, [ra], jnp.where(push, one, zero),
                               mask=act)
            t1 = jnp.where(push, cn, f)
            t2 = jnp.where(push, f, jnp.where(f == 0, one, g))
            plsc.store_scatter(t1_v, [lane_s + cn], t1)
            plsc.store_scatter(t2_v, [lane_s + cn], t2)
            return cn

        lax.fori_loop(0, SEQ, step, zero, unroll=4)
        pltpu.sync_copy(visit_v, visit_hbm.at[pl.ds(b0 * SEQ, NLANE * SEQ)])
        pltpu.sync_copy(t1_v, t1_hbm.at[pl.ds(b0 * SEQ, NLANE * SEQ)])
        pltpu.sync_copy(t2_v, t2_hbm.at[pl.ds(b0 * SEQ, NLANE * SEQ)])


_scan_call = functools.partial(
    pl.kernel,
    out_type=[
        jax.ShapeDtypeStruct((BATCH * SEQ,), jnp.int32),
        jax.ShapeDtypeStruct((BATCH * SEQ,), jnp.int32),
        jax.ShapeDtypeStruct((BATCH * SEQ,), jnp.int32),
    ],
    mesh=plsc.VectorSubcoreMesh(core_axis_name="c", subcore_axis_name="s"),
    compiler_params=pltpu.CompilerParams(needs_layout_passes=False),
    scratch_types=[
        pltpu.VMEM((NLANE * SEQ,), jnp.int32),
        pltpu.VMEM((NLANE * SEQ,), jnp.int32),
        pltpu.VMEM((NLANE * SEQ,), jnp.int32),
        pltpu.VMEM((NLANE * SEQ,), jnp.int32),
        pltpu.VMEM((NNODE * NLANE,), jnp.int32),
        pltpu.VMEM((NNODE * NLANE,), jnp.int32),
        pltpu.VMEM((NNODE * NLANE,), jnp.int32),
    ],
)(_scan_body)


SUB = 16
NSUB = CH // SUB


def _gather_body(idx_hbm, pat_hbm, out_hbm, idx_v, b0_v, b1_v, pat_sh,
                 gsem, osem):
    wid = _widx()
    row0 = wid * ROWS_PER_W

    @pl.when(lax.axis_index("s") == 0)
    def _():
        pltpu.sync_copy(pat_hbm, pat_sh)

    pltpu.sync_copy(idx_hbm.at[pl.ds(row0, ROWS_PER_W)], idx_v)
    plsc.subcore_barrier()

    bufs = [b0_v, b1_v]

    def fire(ci, buf):
        return [
            pltpu.async_copy(
                pat_sh.at[idx_v.at[pl.ds(ci * CH + k * SUB, SUB)]],
                buf.at[pl.ds(k * SUB, SUB)], gsem)
            for k in range(NSUB)
        ]

    g = [None, None]
    o = [None, None]
    g[0] = fire(0, bufs[0])
    for ci in range(NCH):
        cur = ci % 2
        for h in g[cur]:
            h.wait()
        if ci + 1 < NCH:
            if o[1 - cur] is not None:
                o[1 - cur].wait()
            g[1 - cur] = fire(ci + 1, bufs[1 - cur])
        bi = ci // (SEQ // CH)
        s0 = (ci % (SEQ // CH)) * CH
        o[cur] = pltpu.async_copy(
            bufs[cur], out_hbm.at[wid * (ROWS_PER_W // SEQ) + bi,
                                  pl.ds(s0, CH), :], osem)
    o[0].wait()
    o[1].wait()


_gather_call = functools.partial(
    pl.kernel,
    out_type=[jax.ShapeDtypeStruct((BATCH, SEQ, EMB), jnp.float32)],
    mesh=plsc.VectorSubcoreMesh(core_axis_name="c", subcore_axis_name="s"),
    compiler_params=pltpu.CompilerParams(needs_layout_passes=False),
    scratch_types=[
        pltpu.VMEM((ROWS_PER_W,), jnp.int32),
        pltpu.VMEM((CH, EMB), jnp.float32),
        pltpu.VMEM((CH, EMB), jnp.float32),
        pltpu.VMEM_SHARED((SEQ, EMB), jnp.float32),
        pltpu.SemaphoreType.DMA,
        pltpu.SemaphoreType.DMA,
    ],
)(_gather_body)


_BB = 8


def _fea_body(x0_ref, x1_ref, wt_ref, vf_ref, t1_ref, t2_ref,
              o_ref, vm_ref, tp_ref):
    o_ref[:] = (x0_ref[:][:, :, None] * wt_ref[0][None, None, :]
                + x1_ref[:][:, :, None] * wt_ref[1][None, None, :])
    vm_ref[:] = vf_ref[:].reshape(_BB, SEQ)
    t1r = t1_ref[:].reshape(_BB, 1, SEQ)
    t2r = t2_ref[:].reshape(_BB, 1, SEQ)
    tp_ref[:] = jnp.concatenate([t1r, t2r], axis=1)


def _fea_emb(x0, x1, wt, vf, t1f, t2f):
    return pl.pallas_call(
        _fea_body,
        grid=(BATCH // _BB,),
        in_specs=[
            pl.BlockSpec((_BB, SEQ), lambda i: (i, 0)),
            pl.BlockSpec((_BB, SEQ), lambda i: (i, 0)),
            pl.BlockSpec((2, EMB), lambda i: (0, 0)),
            pl.BlockSpec((_BB * SEQ,), lambda i: (i,)),
            pl.BlockSpec((_BB * SEQ,), lambda i: (i,)),
            pl.BlockSpec((_BB * SEQ,), lambda i: (i,)),
        ],
        out_specs=[
            pl.BlockSpec((_BB, SEQ, EMB), lambda i: (i, 0, 0)),
            pl.BlockSpec((_BB, SEQ), lambda i: (i, 0)),
            pl.BlockSpec((_BB, 2, SEQ), lambda i: (i, 0, 0)),
        ],
        out_shape=[
            jax.ShapeDtypeStruct((BATCH, SEQ, EMB), jnp.float32),
            jax.ShapeDtypeStruct((BATCH, SEQ), jnp.int32),
            jax.ShapeDtypeStruct((BATCH, 2, SEQ), jnp.int32),
        ],
    )(x0, x1, wt, vf, t1f, t2f)


def kernel(x, solution, W, pattern, clac_stacks):
    zv = jnp.zeros((NLANE * SEQ,), jnp.int32)
    zl = jnp.zeros((NNODE * NLANE,), jnp.int32)
    visit_flat, t1f, t2f = _scan_call(solution.reshape(-1), zv, zl)

    (pos_emb,) = _gather_call(visit_flat, pattern)

    fea_emb, visit_mod, tp = _fea_emb(
        x[:, :, 0], x[:, :, 1], W.T, visit_flat, t1f, t2f)
    top2_out = tp.transpose(0, 2, 1) * jnp.asarray(clac_stacks, jnp.int32)
    return (fea_emb, pos_emb, visit_mod, top2_out)


# speedup vs baseline: 1.0114x; 1.0114x over previous
import functools

import jax
import jax.numpy as jnp
from jax import lax
from jax.experimental import pallas as pl
from jax.experimental.pallas import tpu as pltpu
from jax.experimental.pallas import tpu_sc as plsc

SEQ = 1024
BATCH = 256
HALF = SEQ // 2
SENT = HALF + 1
NNODE = SENT + 1
EMB = 128
NLANE = 16
NSCAN = BATCH // NLANE
NW = 32
ROWS_PER_W = BATCH * SEQ // NW
CH = 256
NCH = ROWS_PER_W // CH


def _widx():
    return lax.axis_index("s") * 2 + lax.axis_index("c")


def _splat(v):
    return jnp.full((NLANE,), v, jnp.int32)


def _scan_body(sol_hbm, zv_hbm, zl_hbm, visit_hbm, t1_hbm, t2_hbm,
               sol_v, visit_v, t1_v, t2_v, nx_v, pv_v, il_v):
    wid = _widx()

    @pl.when(wid < NSCAN)
    def _():
        b0 = wid * NLANE
        pltpu.sync_copy(sol_hbm.at[pl.ds(b0 * SEQ, NLANE * SEQ)], sol_v)
        pltpu.sync_copy(zv_hbm, visit_v)
        pltpu.sync_copy(zv_hbm, t1_v)
        pltpu.sync_copy(zv_hbm, t2_v)
        pltpu.sync_copy(zl_hbm, il_v)
        nx_v[pl.ds(SENT * NLANE, NLANE)] = _splat(0)
        pv_v[pl.ds(0, NLANE)] = _splat(SENT)

        lane = lax.iota(jnp.int32, NLANE)
        lane_s = lane * SEQ
        sent = _splat(SENT)
        one = _splat(1)
        zero = _splat(0)

        def step(i, pre):
            cn = plsc.load_gather(sol_v, [lane_s + pre])
            vm = jnp.broadcast_to((i + 1) & (SEQ - 1), (NLANE,)).astype(jnp.int32)
            plsc.store_scatter(visit_v, [lane_s + cn], vm)
            push = (cn >= 1) & (cn <= HALF)
            pop = cn > HALF
            act = push | pop
            r = jnp.where(push, cn, cn - HALF)
            ra = r * NLANE + lane
            il = plsc.load_gather(il_v, [ra])
            rem = act & (il > 0)
            rn = plsc.load_gather(nx_v, [ra])
            rp = plsc.load_gather(pv_v, [ra])
            plsc.store_scatter(nx_v, [rp * NLANE + lane], rn, mask=rem)
            plsc.store_scatter(pv_v, [rn * NLANE + lane], rp, mask=rem)
            sa = SENT * NLANE + lane
            f = plsc.load_gather(nx_v, [sa])
            g = plsc.load_gather(nx_v, [f * NLANE + lane])
            ca = cn * NLANE + lane
            plsc.store_scatter(nx_v, [sa], cn, mask=push)
            plsc.store_scatter(pv_v, [ca], sent, mask=push)
            plsc.store_scatter(nx_v, [ca], f, mask=push)
            plsc.store_scatter(pv_v, [f * NLANE + lane], cn, mask=push)
            plsc.store_scatter(il_v, [ra], jnp.where(push, one, zero),
                               mask=act)
            t1 = jnp.where(push, cn, f)
            t2 = jnp.where(push, f, jnp.where(f == 0, one, g))
            plsc.store_scatter(t1_v, [lane_s + cn], t1)
            plsc.store_scatter(t2_v, [lane_s + cn], t2)
            return cn

        lax.fori_loop(0, SEQ, step, zero, unroll=4)
        pltpu.sync_copy(visit_v, visit_hbm.at[pl.ds(b0 * SEQ, NLANE * SEQ)])
        pltpu.sync_copy(t1_v, t1_hbm.at[pl.ds(b0 * SEQ, NLANE * SEQ)])
        pltpu.sync_copy(t2_v, t2_hbm.at[pl.ds(b0 * SEQ, NLANE * SEQ)])


_scan_call = functools.partial(
    pl.kernel,
    out_type=[
        jax.ShapeDtypeStruct((BATCH * SEQ,), jnp.int32),
        jax.ShapeDtypeStruct((BATCH * SEQ,), jnp.int32),
        jax.ShapeDtypeStruct((BATCH * SEQ,), jnp.int32),
    ],
    mesh=plsc.VectorSubcoreMesh(core_axis_name="c", subcore_axis_name="s"),
    compiler_params=pltpu.CompilerParams(needs_layout_passes=False),
    scratch_types=[
        pltpu.VMEM((NLANE * SEQ,), jnp.int32),
        pltpu.VMEM((NLANE * SEQ,), jnp.int32),
        pltpu.VMEM((NLANE * SEQ,), jnp.int32),
        pltpu.VMEM((NLANE * SEQ,), jnp.int32),
        pltpu.VMEM((NNODE * NLANE,), jnp.int32),
        pltpu.VMEM((NNODE * NLANE,), jnp.int32),
        pltpu.VMEM((NNODE * NLANE,), jnp.int32),
    ],
)(_scan_body)


SUB = 64
NSUB = CH // SUB


def _gather_body(idx_hbm, pat_hbm, out_hbm, idx_v, b0_v, b1_v, pat_sh,
                 gsem, osem):
    wid = _widx()
    row0 = wid * ROWS_PER_W

    @pl.when(lax.axis_index("s") == 0)
    def _():
        pltpu.sync_copy(pat_hbm, pat_sh)

    pltpu.sync_copy(idx_hbm.at[pl.ds(row0, ROWS_PER_W)], idx_v)
    plsc.subcore_barrier()

    bufs = [b0_v, b1_v]

    def fire(ci, buf):
        return [
            pltpu.async_copy(
                pat_sh.at[idx_v.at[pl.ds(ci * CH + k * SUB, SUB)]],
                buf.at[pl.ds(k * SUB, SUB)], gsem)
            for k in range(NSUB)
        ]

    g = [None, None]
    o = [None, None]
    g[0] = fire(0, bufs[0])
    for ci in range(NCH):
        cur = ci % 2
        for h in g[cur]:
            h.wait()
        if ci + 1 < NCH:
            if o[1 - cur] is not None:
                o[1 - cur].wait()
            g[1 - cur] = fire(ci + 1, bufs[1 - cur])
        bi = ci // (SEQ // CH)
        s0 = (ci % (SEQ // CH)) * CH
        o[cur] = pltpu.async_copy(
            bufs[cur], out_hbm.at[wid * (ROWS_PER_W // SEQ) + bi,
                                  pl.ds(s0, CH), :], osem)
    o[0].wait()
    o[1].wait()


_gather_call = functools.partial(
    pl.kernel,
    out_type=[jax.ShapeDtypeStruct((BATCH, SEQ, EMB), jnp.float32)],
    mesh=plsc.VectorSubcoreMesh(core_axis_name="c", subcore_axis_name="s"),
    compiler_params=pltpu.CompilerParams(needs_layout_passes=False),
    scratch_types=[
        pltpu.VMEM((ROWS_PER_W,), jnp.int32),
        pltpu.VMEM((CH, EMB), jnp.float32),
        pltpu.VMEM((CH, EMB), jnp.float32),
        pltpu.VMEM_SHARED((SEQ, EMB), jnp.float32),
        pltpu.SemaphoreType.DMA,
        pltpu.SemaphoreType.DMA,
    ],
)(_gather_body)


_BB = 8


def _fea_body(x0_ref, x1_ref, wt_ref, vf_ref, t1_ref, t2_ref,
              o_ref, vm_ref, tp_ref):
    o_ref[:] = (x0_ref[:][:, :, None] * wt_ref[0][None, None, :]
                + x1_ref[:][:, :, None] * wt_ref[1][None, None, :])
    vm_ref[:] = vf_ref[:].reshape(_BB, SEQ)
    t1r = t1_ref[:].reshape(_BB, 1, SEQ)
    t2r = t2_ref[:].reshape(_BB, 1, SEQ)
    tp_ref[:] = jnp.concatenate([t1r, t2r], axis=1)


def _fea_emb(x0, x1, wt, vf, t1f, t2f):
    return pl.pallas_call(
        _fea_body,
        grid=(BATCH // _BB,),
        in_specs=[
            pl.BlockSpec((_BB, SEQ), lambda i: (i, 0)),
            pl.BlockSpec((_BB, SEQ), lambda i: (i, 0)),
            pl.BlockSpec((2, EMB), lambda i: (0, 0)),
            pl.BlockSpec((_BB * SEQ,), lambda i: (i,)),
            pl.BlockSpec((_BB * SEQ,), lambda i: (i,)),
            pl.BlockSpec((_BB * SEQ,), lambda i: (i,)),
        ],
        out_specs=[
            pl.BlockSpec((_BB, SEQ, EMB), lambda i: (i, 0, 0)),
            pl.BlockSpec((_BB, SEQ), lambda i: (i, 0)),
            pl.BlockSpec((_BB, 2, SEQ), lambda i: (i, 0, 0)),
        ],
        out_shape=[
            jax.ShapeDtypeStruct((BATCH, SEQ, EMB), jnp.float32),
            jax.ShapeDtypeStruct((BATCH, SEQ), jnp.int32),
            jax.ShapeDtypeStruct((BATCH, 2, SEQ), jnp.int32),
        ],
    )(x0, x1, wt, vf, t1f, t2f)


def kernel(x, solution, W, pattern, clac_stacks):
    zv = jnp.zeros((NLANE * SEQ,), jnp.int32)
    zl = jnp.zeros((NNODE * NLANE,), jnp.int32)
    visit_flat, t1f, t2f = _scan_call(solution.reshape(-1), zv, zl)

    (pos_emb,) = _gather_call(visit_flat, pattern)

    fea_emb, visit_mod, tp = _fea_emb(
        x[:, :, 0], x[:, :, 1], W.T, visit_flat, t1f, t2f)
    top2_out = tp.transpose(0, 2, 1) * jnp.asarray(clac_stacks, jnp.int32)
    return (fea_emb, pos_emb, visit_mod, top2_out)


# speedup vs baseline: 1.0263x; 1.0148x over previous
import functools

import jax
import jax.numpy as jnp
from jax import lax
from jax.experimental import pallas as pl
from jax.experimental.pallas import tpu as pltpu
from jax.experimental.pallas import tpu_sc as plsc

SEQ = 1024
BATCH = 256
HALF = SEQ // 2
SENT = HALF + 1
NNODE = SENT + 1
EMB = 128
NLANE = 16
NSCAN = BATCH // NLANE
NW = 32
ROWS_PER_W = BATCH * SEQ // NW
CH = 256
NCH = ROWS_PER_W // CH


def _widx():
    return lax.axis_index("s") * 2 + lax.axis_index("c")


def _splat(v):
    return jnp.full((NLANE,), v, jnp.int32)


def _scan_body(sol_hbm, zv_hbm, zl_hbm, visit_hbm, t1_hbm, t2_hbm,
               sol_v, visit_v, t1_v, t2_v, nx_v, pv_v, il_v):
    wid = _widx()

    @pl.when(wid < NSCAN)
    def _():
        b0 = wid * NLANE
        pltpu.sync_copy(sol_hbm.at[pl.ds(b0 * SEQ, NLANE * SEQ)], sol_v)
        pltpu.sync_copy(zv_hbm, visit_v)
        pltpu.sync_copy(zv_hbm, t1_v)
        pltpu.sync_copy(zv_hbm, t2_v)
        pltpu.sync_copy(zl_hbm, il_v)
        nx_v[pl.ds(SENT * NLANE, NLANE)] = _splat(0)
        pv_v[pl.ds(0, NLANE)] = _splat(SENT)

        lane = lax.iota(jnp.int32, NLANE)
        lane_s = lane * SEQ
        sent = _splat(SENT)
        one = _splat(1)
        zero = _splat(0)

        def step(i, pre):
            cn = plsc.load_gather(sol_v, [lane_s + pre])
            vm = jnp.broadcast_to((i + 1) & (SEQ - 1), (NLANE,)).astype(jnp.int32)
            plsc.store_scatter(visit_v, [lane_s + cn], vm)
            push = (cn >= 1) & (cn <= HALF)
            pop = cn > HALF
            act = push | pop
            r = jnp.where(push, cn, cn - HALF)
            ra = r * NLANE + lane
            il = plsc.load_gather(il_v, [ra])
            rem = act & (il > 0)
            rn = plsc.load_gather(nx_v, [ra])
            rp = plsc.load_gather(pv_v, [ra])
            plsc.store_scatter(nx_v, [rp * NLANE + lane], rn, mask=rem)
            plsc.store_scatter(pv_v, [rn * NLANE + lane], rp, mask=rem)
            sa = SENT * NLANE + lane
            f = plsc.load_gather(nx_v, [sa])
            g = plsc.load_gather(nx_v, [f * NLANE + lane])
            ca = cn * NLANE + lane
            plsc.store_scatter(nx_v, [sa], cn, mask=push)
            plsc.store_scatter(pv_v, [ca], sent, mask=push)
            plsc.store_scatter(nx_v, [ca], f, mask=push)
            plsc.store_scatter(pv_v, [f * NLANE + lane], cn, mask=push)
            plsc.store_scatter(il_v, [ra], jnp.where(push, one, zero),
                               mask=act)
            t1 = jnp.where(push, cn, f)
            t2 = jnp.where(push, f, jnp.where(f == 0, one, g))
            plsc.store_scatter(t1_v, [lane_s + cn], t1)
            plsc.store_scatter(t2_v, [lane_s + cn], t2)
            return cn

        lax.fori_loop(0, SEQ, step, zero, unroll=4)
        pltpu.sync_copy(visit_v, visit_hbm.at[pl.ds(b0 * SEQ, NLANE * SEQ)])
        pltpu.sync_copy(t1_v, t1_hbm.at[pl.ds(b0 * SEQ, NLANE * SEQ)])
        pltpu.sync_copy(t2_v, t2_hbm.at[pl.ds(b0 * SEQ, NLANE * SEQ)])


_scan_call = functools.partial(
    pl.kernel,
    out_type=[
        jax.ShapeDtypeStruct((BATCH * SEQ,), jnp.int32),
        jax.ShapeDtypeStruct((BATCH * SEQ,), jnp.int32),
        jax.ShapeDtypeStruct((BATCH * SEQ,), jnp.int32),
    ],
    mesh=plsc.VectorSubcoreMesh(core_axis_name="c", subcore_axis_name="s"),
    compiler_params=pltpu.CompilerParams(needs_layout_passes=False),
    scratch_types=[
        pltpu.VMEM((NLANE * SEQ,), jnp.int32),
        pltpu.VMEM((NLANE * SEQ,), jnp.int32),
        pltpu.VMEM((NLANE * SEQ,), jnp.int32),
        pltpu.VMEM((NLANE * SEQ,), jnp.int32),
        pltpu.VMEM((NNODE * NLANE,), jnp.int32),
        pltpu.VMEM((NNODE * NLANE,), jnp.int32),
        pltpu.VMEM((NNODE * NLANE,), jnp.int32),
    ],
)(_scan_body)


SUB = 32
NSUB = CH // SUB


def _gather_body(idx_hbm, pat_hbm, out_hbm, idx_v, b0_v, b1_v, pat_sh,
                 gsem, osem):
    wid = _widx()
    row0 = wid * ROWS_PER_W

    @pl.when(lax.axis_index("s") == 0)
    def _():
        pltpu.sync_copy(pat_hbm, pat_sh)

    pltpu.sync_copy(idx_hbm.at[pl.ds(row0, ROWS_PER_W)], idx_v)
    plsc.subcore_barrier()

    bufs = [b0_v, b1_v]

    def fire(ci, buf):
        return [
            pltpu.async_copy(
                pat_sh.at[idx_v.at[pl.ds(ci * CH + k * SUB, SUB)]],
                buf.at[pl.ds(k * SUB, SUB)], gsem)
            for k in range(NSUB)
        ]

    g = [None, None]
    o = [None, None]
    g[0] = fire(0, bufs[0])
    for ci in range(NCH):
        cur = ci % 2
        for h in g[cur]:
            h.wait()
        if ci + 1 < NCH:
            if o[1 - cur] is not None:
                o[1 - cur].wait()
            g[1 - cur] = fire(ci + 1, bufs[1 - cur])
        bi = ci // (SEQ // CH)
        s0 = (ci % (SEQ // CH)) * CH
        o[cur] = pltpu.async_copy(
            bufs[cur], out_hbm.at[wid * (ROWS_PER_W // SEQ) + bi,
                                  pl.ds(s0, CH), :], osem)
    o[0].wait()
    o[1].wait()


_gather_call = functools.partial(
    pl.kernel,
    out_type=[jax.ShapeDtypeStruct((BATCH, SEQ, EMB), jnp.float32)],
    mesh=plsc.VectorSubcoreMesh(core_axis_name="c", subcore_axis_name="s"),
    compiler_params=pltpu.CompilerParams(needs_layout_passes=False),
    scratch_types=[
        pltpu.VMEM((ROWS_PER_W,), jnp.int32),
        pltpu.VMEM((CH, EMB), jnp.float32),
        pltpu.VMEM((CH, EMB), jnp.float32),
        pltpu.VMEM_SHARED((SEQ, EMB), jnp.float32),
        pltpu.SemaphoreType.DMA,
        pltpu.SemaphoreType.DMA,
    ],
)(_gather_body)


_BB = 8


def _fea_body(x0_ref, x1_ref, wt_ref, vf_ref, t1_ref, t2_ref,
              o_ref, vm_ref, tp_ref):
    o_ref[:] = (x0_ref[:][:, :, None] * wt_ref[0][None, None, :]
                + x1_ref[:][:, :, None] * wt_ref[1][None, None, :])
    vm_ref[:] = vf_ref[:].reshape(_BB, SEQ)
    t1r = t1_ref[:].reshape(_BB, 1, SEQ)
    t2r = t2_ref[:].reshape(_BB, 1, SEQ)
    tp_ref[:] = jnp.concatenate([t1r, t2r], axis=1)


def _fea_emb(x0, x1, wt, vf, t1f, t2f):
    return pl.pallas_call(
        _fea_body,
        grid=(BATCH // _BB,),
        in_specs=[
            pl.BlockSpec((_BB, SEQ), lambda i: (i, 0)),
            pl.BlockSpec((_BB, SEQ), lambda i: (i, 0)),
            pl.BlockSpec((2, EMB), lambda i: (0, 0)),
            pl.BlockSpec((_BB * SEQ,), lambda i: (i,)),
            pl.BlockSpec((_BB * SEQ,), lambda i: (i,)),
            pl.BlockSpec((_BB * SEQ,), lambda i: (i,)),
        ],
        out_specs=[
            pl.BlockSpec((_BB, SEQ, EMB), lambda i: (i, 0, 0)),
            pl.BlockSpec((_BB, SEQ), lambda i: (i, 0)),
            pl.BlockSpec((_BB, 2, SEQ), lambda i: (i, 0, 0)),
        ],
        out_shape=[
            jax.ShapeDtypeStruct((BATCH, SEQ, EMB), jnp.float32),
            jax.ShapeDtypeStruct((BATCH, SEQ), jnp.int32),
            jax.ShapeDtypeStruct((BATCH, 2, SEQ), jnp.int32),
        ],
    )(x0, x1, wt, vf, t1f, t2f)


def kernel(x, solution, W, pattern, clac_stacks):
    zv = jnp.zeros((NLANE * SEQ,), jnp.int32)
    zl = jnp.zeros((NNODE * NLANE,), jnp.int32)
    visit_flat, t1f, t2f = _scan_call(solution.reshape(-1), zv, zl)

    (pos_emb,) = _gather_call(visit_flat, pattern)

    fea_emb, visit_mod, tp = _fea_emb(
        x[:, :, 0], x[:, :, 1], W.T, visit_flat, t1f, t2f)
    top2_out = tp.transpose(0, 2, 1) * jnp.asarray(clac_stacks, jnp.int32)
    return (fea_emb, pos_emb, visit_mod, top2_out)


# speedup vs baseline: 1.0266x; 1.0003x over previous
import functools

import jax
import jax.numpy as jnp
from jax import lax
from jax.experimental import pallas as pl
from jax.experimental.pallas import tpu as pltpu
from jax.experimental.pallas import tpu_sc as plsc

SEQ = 1024
BATCH = 256
HALF = SEQ // 2
SENT = HALF + 1
NNODE = SENT + 1
EMB = 128
NLANE = 16
NSCAN = BATCH // NLANE
NW = 32
ROWS_PER_W = BATCH * SEQ // NW
CH = 256
NCH = ROWS_PER_W // CH


def _widx():
    return lax.axis_index("s") * 2 + lax.axis_index("c")


def _splat(v):
    return jnp.full((NLANE,), v, jnp.int32)


def _scan_body(sol_hbm, zv_hbm, zl_hbm, visit_hbm, t1_hbm, t2_hbm,
               sol_v, visit_v, t1_v, t2_v, nx_v, pv_v, il_v):
    wid = _widx()

    @pl.when(wid < NSCAN)
    def _():
        b0 = wid * NLANE
        pltpu.sync_copy(sol_hbm.at[pl.ds(b0 * SEQ, NLANE * SEQ)], sol_v)
        pltpu.sync_copy(zv_hbm, visit_v)
        pltpu.sync_copy(zv_hbm, t1_v)
        pltpu.sync_copy(zv_hbm, t2_v)
        pltpu.sync_copy(zl_hbm, il_v)
        nx_v[pl.ds(SENT * NLANE, NLANE)] = _splat(0)
        pv_v[pl.ds(0, NLANE)] = _splat(SENT)

        lane = lax.iota(jnp.int32, NLANE)
        lane_s = lane * SEQ
        sent = _splat(SENT)
        one = _splat(1)
        zero = _splat(0)

        def step(i, pre):
            cn = plsc.load_gather(sol_v, [lane_s + pre])
            vm = jnp.broadcast_to((i + 1) & (SEQ - 1), (NLANE,)).astype(jnp.int32)
            plsc.store_scatter(visit_v, [lane_s + cn], vm)
            push = (cn >= 1) & (cn <= HALF)
            pop = cn > HALF
            act = push | pop
            r = jnp.where(push, cn, cn - HALF)
            ra = r * NLANE + lane
            il = plsc.load_gather(il_v, [ra])
            rem = act & (il > 0)
            rn = plsc.load_gather(nx_v, [ra])
            rp = plsc.load_gather(pv_v, [ra])
            plsc.store_scatter(nx_v, [rp * NLANE + lane], rn, mask=rem)
            plsc.store_scatter(pv_v, [rn * NLANE + lane], rp, mask=rem)
            sa = SENT * NLANE + lane
            f = plsc.load_gather(nx_v, [sa])
            g = plsc.load_gather(nx_v, [f * NLANE + lane])
            ca = cn * NLANE + lane
            plsc.store_scatter(nx_v, [sa], cn, mask=push)
            plsc.store_scatter(pv_v, [ca], sent, mask=push)
            plsc.store_scatter(nx_v, [ca], f, mask=push)
            plsc.store_scatter(pv_v, [f * NLANE + lane], cn, mask=push)
            plsc.store_scatter(il_v, [ra], jnp.where(push, one, zero),
                               mask=act)
            t1 = jnp.where(push, cn, f)
            t2 = jnp.where(push, f, jnp.where(f == 0, one, g))
            plsc.store_scatter(t1_v, [lane_s + cn], t1)
            plsc.store_scatter(t2_v, [lane_s + cn], t2)
            return cn

        lax.fori_loop(0, SEQ, step, zero, unroll=4)
        pltpu.sync_copy(visit_v, visit_hbm.at[pl.ds(b0 * SEQ, NLANE * SEQ)])
        pltpu.sync_copy(t1_v, t1_hbm.at[pl.ds(b0 * SEQ, NLANE * SEQ)])
        pltpu.sync_copy(t2_v, t2_hbm.at[pl.ds(b0 * SEQ, NLANE * SEQ)])


_scan_call = functools.partial(
    pl.kernel,
    out_type=[
        jax.ShapeDtypeStruct((BATCH * SEQ,), jnp.int32),
        jax.ShapeDtypeStruct((BATCH * SEQ,), jnp.int32),
        jax.ShapeDtypeStruct((BATCH * SEQ,), jnp.int32),
    ],
    mesh=plsc.VectorSubcoreMesh(core_axis_name="c", subcore_axis_name="s"),
    compiler_params=pltpu.CompilerParams(needs_layout_passes=False),
    scratch_types=[
        pltpu.VMEM((NLANE * SEQ,), jnp.int32),
        pltpu.VMEM((NLANE * SEQ,), jnp.int32),
        pltpu.VMEM((NLANE * SEQ,), jnp.int32),
        pltpu.VMEM((NLANE * SEQ,), jnp.int32),
        pltpu.VMEM((NNODE * NLANE,), jnp.int32),
        pltpu.VMEM((NNODE * NLANE,), jnp.int32),
        pltpu.VMEM((NNODE * NLANE,), jnp.int32),
    ],
)(_scan_body)


SUB = 32
NSUB = CH // SUB


def _gather_body(idx_hbm, pat_hbm, out_hbm, idx_v, b0_v, b1_v, pat_sh,
                 gsem, osem):
    wid = _widx()
    row0 = wid * ROWS_PER_W

    @pl.when(lax.axis_index("s") == 0)
    def _():
        pltpu.sync_copy(pat_hbm, pat_sh)

    pltpu.sync_copy(idx_hbm.at[pl.ds(row0, ROWS_PER_W)], idx_v)
    plsc.subcore_barrier()

    bufs = [b0_v, b1_v]

    def fire(ci, buf):
        return [
            pltpu.async_copy(
                pat_sh.at[idx_v.at[pl.ds(ci * CH + k * SUB, SUB)]],
                buf.at[pl.ds(k * SUB, SUB)], gsem)
            for k in range(NSUB)
        ]

    g = [None, None]
    o = [None, None]
    g[0] = fire(0, bufs[0])
    for ci in range(NCH):
        cur = ci % 2
        for h in g[cur]:
            h.wait()
        if ci + 1 < NCH:
            if o[1 - cur] is not None:
                o[1 - cur].wait()
            g[1 - cur] = fire(ci + 1, bufs[1 - cur])
        bi = ci // (SEQ // CH)
        s0 = (ci % (SEQ // CH)) * CH
        o[cur] = pltpu.async_copy(
            bufs[cur], out_hbm.at[wid * (ROWS_PER_W // SEQ) + bi,
                                  pl.ds(s0, CH), :], osem)
    o[0].wait()
    o[1].wait()


_gather_call = functools.partial(
    pl.kernel,
    out_type=[jax.ShapeDtypeStruct((BATCH, SEQ, EMB), jnp.float32)],
    mesh=plsc.VectorSubcoreMesh(core_axis_name="c", subcore_axis_name="s"),
    compiler_params=pltpu.CompilerParams(needs_layout_passes=False),
    scratch_types=[
        pltpu.VMEM((ROWS_PER_W,), jnp.int32),
        pltpu.VMEM((CH, EMB), jnp.float32),
        pltpu.VMEM((CH, EMB), jnp.float32),
        pltpu.VMEM_SHARED((SEQ, EMB), jnp.float32),
        pltpu.SemaphoreType.DMA,
        pltpu.SemaphoreType.DMA,
    ],
)(_gather_body)


_BB = 8


def _fea_body(x0_ref, x1_ref, wt_ref, o_ref):
    o_ref[:] = (x0_ref[:][:, :, None] * wt_ref[0][None, None, :]
                + x1_ref[:][:, :, None] * wt_ref[1][None, None, :])


def _fea_emb(x0, x1, wt):
    return pl.pallas_call(
        _fea_body,
        grid=(BATCH // _BB,),
        in_specs=[
            pl.BlockSpec((_BB, SEQ), lambda i: (i, 0)),
            pl.BlockSpec((_BB, SEQ), lambda i: (i, 0)),
            pl.BlockSpec((2, EMB), lambda i: (0, 0)),
        ],
        out_specs=pl.BlockSpec((_BB, SEQ, EMB), lambda i: (i, 0, 0)),
        out_shape=jax.ShapeDtypeStruct((BATCH, SEQ, EMB), jnp.float32),
    )(x0, x1, wt)


_FB = 32


def _fmt_body(vf_ref, t1_ref, t2_ref, vm_ref, tp_ref):
    vm_ref[:] = vf_ref[:].reshape(_FB, SEQ)
    t1r = t1_ref[:].reshape(_FB, 1, SEQ)
    t2r = t2_ref[:].reshape(_FB, 1, SEQ)
    tp_ref[:] = jnp.concatenate([t1r, t2r], axis=1)


def _fmt(vf, t1f, t2f):
    return pl.pallas_call(
        _fmt_body,
        grid=(BATCH // _FB,),
        in_specs=[
            pl.BlockSpec((_FB * SEQ,), lambda i: (i,)),
            pl.BlockSpec((_FB * SEQ,), lambda i: (i,)),
            pl.BlockSpec((_FB * SEQ,), lambda i: (i,)),
        ],
        out_specs=[
            pl.BlockSpec((_FB, SEQ), lambda i: (i, 0)),
            pl.BlockSpec((_FB, 2, SEQ), lambda i: (i, 0, 0)),
        ],
        out_shape=[
            jax.ShapeDtypeStruct((BATCH, SEQ), jnp.int32),
            jax.ShapeDtypeStruct((BATCH, 2, SEQ), jnp.int32),
        ],
    )(vf, t1f, t2f)


def kernel(x, solution, W, pattern, clac_stacks):
    zv = jnp.zeros((NLANE * SEQ,), jnp.int32)
    zl = jnp.zeros((NNODE * NLANE,), jnp.int32)
    visit_flat, t1f, t2f = _scan_call(solution.reshape(-1), zv, zl)

    (pos_emb,) = _gather_call(visit_flat, pattern)

    fea_emb = _fea_emb(x[:, :, 0], x[:, :, 1], W.T)
    visit_mod, tp = _fmt(visit_flat, t1f, t2f)
    top2_out = tp.transpose(0, 2, 1) * jnp.asarray(clac_stacks, jnp.int32)
    return (fea_emb, pos_emb, visit_mod, top2_out)


# speedup vs baseline: 1.0644x; 1.0369x over previous
import functools

import jax
import jax.numpy as jnp
from jax import lax
from jax.experimental import pallas as pl
from jax.experimental.pallas import tpu as pltpu
from jax.experimental.pallas import tpu_sc as plsc

SEQ = 1024
BATCH = 256
HALF = SEQ // 2
SENT = HALF + 1
NNODE = SENT + 1
EMB = 128
NLANE = 16
NSCAN = BATCH // NLANE
NW = 32
ROWS_PER_W = BATCH * SEQ // NW
CH = 256
NCH = ROWS_PER_W // CH


def _widx():
    return lax.axis_index("s") * 2 + lax.axis_index("c")


def _splat(v):
    return jnp.full((NLANE,), v, jnp.int32)


def _scan_body(sol_hbm, zv_hbm, zl_hbm, visit_hbm, t1_hbm, t2_hbm,
               sol_v, visit_v, t1_v, t2_v, nx_v, pv_v, il_v):
    wid = _widx()

    @pl.when(wid < NSCAN)
    def _():
        b0 = wid * NLANE
        pltpu.sync_copy(sol_hbm.at[pl.ds(b0 * SEQ, NLANE * SEQ)], sol_v)
        pltpu.sync_copy(zv_hbm, visit_v)
        pltpu.sync_copy(zv_hbm, t1_v)
        pltpu.sync_copy(zv_hbm, t2_v)
        pltpu.sync_copy(zl_hbm, il_v)
        nx_v[pl.ds(SENT * NLANE, NLANE)] = _splat(0)
        pv_v[pl.ds(0, NLANE)] = _splat(SENT)

        lane = lax.iota(jnp.int32, NLANE)
        lane_s = lane * SEQ
        sent = _splat(SENT)
        one = _splat(1)
        zero = _splat(0)

        def step(i, carry):
            pre, h, s = carry
            cn = plsc.load_gather(sol_v, [lane_s + pre])
            vm = jnp.broadcast_to((i + 1) & (SEQ - 1), (NLANE,)).astype(jnp.int32)
            plsc.store_scatter(visit_v, [lane_s + cn], vm)
            push = (cn >= 1) & (cn <= HALF)
            pop = cn > HALF
            act = push | pop
            r = jnp.where(push, cn, cn - HALF)
            ra = r * NLANE + lane
            il = plsc.load_gather(il_v, [ra])
            rn = plsc.load_gather(nx_v, [ra])
            rp = plsc.load_gather(pv_v, [ra])
            g = plsc.load_gather(nx_v, [s * NLANE + lane])
            rem = act & (il > 0)
            plsc.store_scatter(nx_v, [rp * NLANE + lane], rn, mask=rem)
            plsc.store_scatter(pv_v, [rn * NLANE + lane], rp, mask=rem)
            hr = jnp.where(rem & (cn == h), s, h)
            ca = cn * NLANE + lane
            plsc.store_scatter(nx_v, [SENT * NLANE + lane], cn, mask=push)
            plsc.store_scatter(pv_v, [ca], sent, mask=push)
            plsc.store_scatter(nx_v, [ca], hr, mask=push)
            plsc.store_scatter(pv_v, [hr * NLANE + lane], cn, mask=push)
            plsc.store_scatter(il_v, [ra], jnp.where(push, one, zero),
                               mask=act)
            poph = pop & rem & (r == h)
            pops = pop & rem & (r == s)
            h_new = jnp.where(push, cn, jnp.where(poph, s, h))
            s_new = jnp.where(
                push, hr,
                jnp.where(poph, jnp.where(s == 0, one, g),
                          jnp.where(pops, g, s)))
            t2 = jnp.where(h_new == 0, one, s_new)
            plsc.store_scatter(t1_v, [lane_s + cn], h_new)
            plsc.store_scatter(t2_v, [lane_s + cn], t2)
            return (cn, h_new, s_new)

        lax.fori_loop(0, SEQ, step, (zero, zero, one), unroll=4)
        pltpu.sync_copy(visit_v, visit_hbm.at[pl.ds(b0 * SEQ, NLANE * SEQ)])
        pltpu.sync_copy(t1_v, t1_hbm.at[pl.ds(b0 * SEQ, NLANE * SEQ)])
        pltpu.sync_copy(t2_v, t2_hbm.at[pl.ds(b0 * SEQ, NLANE * SEQ)])


_scan_call = functools.partial(
    pl.kernel,
    out_type=[
        jax.ShapeDtypeStruct((BATCH * SEQ,), jnp.int32),
        jax.ShapeDtypeStruct((BATCH * SEQ,), jnp.int32),
        jax.ShapeDtypeStruct((BATCH * SEQ,), jnp.int32),
    ],
    mesh=plsc.VectorSubcoreMesh(core_axis_name="c", subcore_axis_name="s"),
    compiler_params=pltpu.CompilerParams(needs_layout_passes=False),
    scratch_types=[
        pltpu.VMEM((NLANE * SEQ,), jnp.int32),
        pltpu.VMEM((NLANE * SEQ,), jnp.int32),
        pltpu.VMEM((NLANE * SEQ,), jnp.int32),
        pltpu.VMEM((NLANE * SEQ,), jnp.int32),
        pltpu.VMEM((NNODE * NLANE,), jnp.int32),
        pltpu.VMEM((NNODE * NLANE,), jnp.int32),
        pltpu.VMEM((NNODE * NLANE,), jnp.int32),
    ],
)(_scan_body)


SUB = 32
NSUB = CH // SUB


def _gather_body(idx_hbm, pat_hbm, out_hbm, idx_v, b0_v, b1_v, pat_sh,
                 gsem, osem):
    wid = _widx()
    row0 = wid * ROWS_PER_W

    @pl.when(lax.axis_index("s") == 0)
    def _():
        pltpu.sync_copy(pat_hbm, pat_sh)

    pltpu.sync_copy(idx_hbm.at[pl.ds(row0, ROWS_PER_W)], idx_v)
    plsc.subcore_barrier()

    bufs = [b0_v, b1_v]

    def fire(ci, buf):
        return [
            pltpu.async_copy(
                pat_sh.at[idx_v.at[pl.ds(ci * CH + k * SUB, SUB)]],
                buf.at[pl.ds(k * SUB, SUB)], gsem)
            for k in range(NSUB)
        ]

    g = [None, None]
    o = [None, None]
    g[0] = fire(0, bufs[0])
    for ci in range(NCH):
        cur = ci % 2
        for h in g[cur]:
            h.wait()
        if ci + 1 < NCH:
            if o[1 - cur] is not None:
                o[1 - cur].wait()
            g[1 - cur] = fire(ci + 1, bufs[1 - cur])
        bi = ci // (SEQ // CH)
        s0 = (ci % (SEQ // CH)) * CH
        o[cur] = pltpu.async_copy(
            bufs[cur], out_hbm.at[wid * (ROWS_PER_W // SEQ) + bi,
                                  pl.ds(s0, CH), :], osem)
    o[0].wait()
    o[1].wait()


_gather_call = functools.partial(
    pl.kernel,
    out_type=[jax.ShapeDtypeStruct((BATCH, SEQ, EMB), jnp.float32)],
    mesh=plsc.VectorSubcoreMesh(core_axis_name="c", subcore_axis_name="s"),
    compiler_params=pltpu.CompilerParams(needs_layout_passes=False),
    scratch_types=[
        pltpu.VMEM((ROWS_PER_W,), jnp.int32),
        pltpu.VMEM((CH, EMB), jnp.float32),
        pltpu.VMEM((CH, EMB), jnp.float32),
        pltpu.VMEM_SHARED((SEQ, EMB), jnp.float32),
        pltpu.SemaphoreType.DMA,
        pltpu.SemaphoreType.DMA,
    ],
)(_gather_body)


_BB = 8


def _fea_body(x0_ref, x1_ref, wt_ref, o_ref):
    o_ref[:] = (x0_ref[:][:, :, None] * wt_ref[0][None, None, :]
                + x1_ref[:][:, :, None] * wt_ref[1][None, None, :])


def _fea_emb(x0, x1, wt):
    return pl.pallas_call(
        _fea_body,
        grid=(BATCH // _BB,),
        in_specs=[
            pl.BlockSpec((_BB, SEQ), lambda i: (i, 0)),
            pl.BlockSpec((_BB, SEQ), lambda i: (i, 0)),
            pl.BlockSpec((2, EMB), lambda i: (0, 0)),
        ],
        out_specs=pl.BlockSpec((_BB, SEQ, EMB), lambda i: (i, 0, 0)),
        out_shape=jax.ShapeDtypeStruct((BATCH, SEQ, EMB), jnp.float32),
    )(x0, x1, wt)


_FB = 32


def _fmt_body(vf_ref, t1_ref, t2_ref, vm_ref, tp_ref):
    vm_ref[:] = vf_ref[:].reshape(_FB, SEQ)
    t1r = t1_ref[:].reshape(_FB, 1, SEQ)
    t2r = t2_ref[:].reshape(_FB, 1, SEQ)
    tp_ref[:] = jnp.concatenate([t1r, t2r], axis=1)


def _fmt(vf, t1f, t2f):
    return pl.pallas_call(
        _fmt_body,
        grid=(BATCH // _FB,),
        in_specs=[
            pl.BlockSpec((_FB * SEQ,), lambda i: (i,)),
            pl.BlockSpec((_FB * SEQ,), lambda i: (i,)),
            pl.BlockSpec((_FB * SEQ,), lambda i: (i,)),
        ],
        out_specs=[
            pl.BlockSpec((_FB, SEQ), lambda i: (i, 0)),
            pl.BlockSpec((_FB, 2, SEQ), lambda i: (i, 0, 0)),
        ],
        out_shape=[
            jax.ShapeDtypeStruct((BATCH, SEQ), jnp.int32),
            jax.ShapeDtypeStruct((BATCH, 2, SEQ), jnp.int32),
        ],
    )(vf, t1f, t2f)


def kernel(x, solution, W, pattern, clac_stacks):
    zv = jnp.zeros((NLANE * SEQ,), jnp.int32)
    zl = jnp.zeros((NNODE * NLANE,), jnp.int32)
    visit_flat, t1f, t2f = _scan_call(solution.reshape(-1), zv, zl)

    (pos_emb,) = _gather_call(visit_flat, pattern)

    fea_emb = _fea_emb(x[:, :, 0], x[:, :, 1], W.T)
    visit_mod, tp = _fmt(visit_flat, t1f, t2f)
    top2_out = tp.transpose(0, 2, 1) * jnp.asarray(clac_stacks, jnp.int32)
    return (fea_emb, pos_emb, visit_mod, top2_out)


# speedup vs baseline: 1.0647x; 1.0002x over previous
import functools

import jax
import jax.numpy as jnp
from jax import lax
from jax.experimental import pallas as pl
from jax.experimental.pallas import tpu as pltpu
from jax.experimental.pallas import tpu_sc as plsc

SEQ = 1024
BATCH = 256
HALF = SEQ // 2
SENT = HALF + 1
NNODE = SENT + 1
EMB = 128
NLANE = 16
NSCAN = BATCH // NLANE
NW = 32
ROWS_PER_W = BATCH * SEQ // NW
CH = 256
NCH = ROWS_PER_W // CH


def _widx():
    return lax.axis_index("s") * 2 + lax.axis_index("c")


def _splat(v):
    return jnp.full((NLANE,), v, jnp.int32)


def _scan_body(sol_hbm, zv_hbm, zl_hbm, visit_hbm, t1_hbm, t2_hbm,
               sol_v, visit_v, t1_v, t2_v, nx_v, pv_v, il_v):
    wid = _widx()

    @pl.when(wid < NSCAN)
    def _():
        b0 = wid * NLANE
        pltpu.sync_copy(sol_hbm.at[pl.ds(b0 * SEQ, NLANE * SEQ)], sol_v)
        pltpu.sync_copy(zv_hbm, visit_v)
        pltpu.sync_copy(zv_hbm, t1_v)
        pltpu.sync_copy(zv_hbm, t2_v)
        pltpu.sync_copy(zl_hbm, il_v)
        nx_v[pl.ds(SENT * NLANE, NLANE)] = _splat(0)
        pv_v[pl.ds(0, NLANE)] = _splat(SENT)

        lane = lax.iota(jnp.int32, NLANE)
        lane_s = lane * SEQ
        sent = _splat(SENT)
        one = _splat(1)
        zero = _splat(0)

        def step(i, carry):
            pre, h, s = carry
            cn = plsc.load_gather(sol_v, [lane_s + pre])
            vm = jnp.broadcast_to((i + 1) & (SEQ - 1), (NLANE,)).astype(jnp.int32)
            plsc.store_scatter(visit_v, [lane_s + cn], vm)
            push = (cn >= 1) & (cn <= HALF)
            pop = cn > HALF
            act = push | pop
            r = jnp.where(push, cn, cn - HALF)
            ra = r * NLANE + lane
            il = plsc.load_gather(il_v, [ra])
            rn = plsc.load_gather(nx_v, [ra])
            rp = plsc.load_gather(pv_v, [ra])
            g = plsc.load_gather(nx_v, [s * NLANE + lane])
            rem = act & (il > 0)
            plsc.store_scatter(nx_v, [rp * NLANE + lane], rn, mask=rem)
            plsc.store_scatter(pv_v, [rn * NLANE + lane], rp, mask=rem)
            hr = jnp.where(rem & (cn == h), s, h)
            ca = cn * NLANE + lane
            plsc.store_scatter(nx_v, [SENT * NLANE + lane], cn, mask=push)
            plsc.store_scatter(pv_v, [ca], sent, mask=push)
            plsc.store_scatter(nx_v, [ca], hr, mask=push)
            plsc.store_scatter(pv_v, [hr * NLANE + lane], cn, mask=push)
            plsc.store_scatter(il_v, [ra], jnp.where(push, one, zero),
                               mask=act)
            poph = pop & rem & (r == h)
            pops = pop & rem & (r == s)
            h_new = jnp.where(push, cn, jnp.where(poph, s, h))
            s_new = jnp.where(
                push, hr,
                jnp.where(poph, jnp.where(s == 0, one, g),
                          jnp.where(pops, g, s)))
            t2 = jnp.where(h_new == 0, one, s_new)
            plsc.store_scatter(t1_v, [lane_s + cn], h_new)
            plsc.store_scatter(t2_v, [lane_s + cn], t2)
            return (cn, h_new, s_new)

        lax.fori_loop(0, SEQ, step, (zero, zero, one), unroll=8)
        pltpu.sync_copy(visit_v, visit_hbm.at[pl.ds(b0 * SEQ, NLANE * SEQ)])
        pltpu.sync_copy(t1_v, t1_hbm.at[pl.ds(b0 * SEQ, NLANE * SEQ)])
        pltpu.sync_copy(t2_v, t2_hbm.at[pl.ds(b0 * SEQ, NLANE * SEQ)])


_scan_call = functools.partial(
    pl.kernel,
    out_type=[
        jax.ShapeDtypeStruct((BATCH * SEQ,), jnp.int32),
        jax.ShapeDtypeStruct((BATCH * SEQ,), jnp.int32),
        jax.ShapeDtypeStruct((BATCH * SEQ,), jnp.int32),
    ],
    mesh=plsc.VectorSubcoreMesh(core_axis_name="c", subcore_axis_name="s"),
    compiler_params=pltpu.CompilerParams(needs_layout_passes=False),
    scratch_types=[
        pltpu.VMEM((NLANE * SEQ,), jnp.int32),
        pltpu.VMEM((NLANE * SEQ,), jnp.int32),
        pltpu.VMEM((NLANE * SEQ,), jnp.int32),
        pltpu.VMEM((NLANE * SEQ,), jnp.int32),
        pltpu.VMEM((NNODE * NLANE,), jnp.int32),
        pltpu.VMEM((NNODE * NLANE,), jnp.int32),
        pltpu.VMEM((NNODE * NLANE,), jnp.int32),
    ],
)(_scan_body)


SUB = 32
NSUB = CH // SUB


def _gather_body(idx_hbm, pat_hbm, out_hbm, idx_v, b0_v, b1_v, pat_sh,
                 gsem, osem):
    wid = _widx()
    row0 = wid * ROWS_PER_W

    @pl.when(lax.axis_index("s") == 0)
    def _():
        pltpu.sync_copy(pat_hbm, pat_sh)

    pltpu.sync_copy(idx_hbm.at[pl.ds(row0, ROWS_PER_W)], idx_v)
    plsc.subcore_barrier()

    bufs = [b0_v, b1_v]

    def fire(ci, buf):
        return [
            pltpu.async_copy(
                pat_sh.at[idx_v.at[pl.ds(ci * CH + k * SUB, SUB)]],
                buf.at[pl.ds(k * SUB, SUB)], gsem)
            for k in range(NSUB)
        ]

    g = [None, None]
    o = [None, None]
    g[0] = fire(0, bufs[0])
    for ci in range(NCH):
        cur = ci % 2
        for h in g[cur]:
            h.wait()
        if ci + 1 < NCH:
            if o[1 - cur] is not None:
                o[1 - cur].wait()
            g[1 - cur] = fire(ci + 1, bufs[1 - cur])
        bi = ci // (SEQ // CH)
        s0 = (ci % (SEQ // CH)) * CH
        o[cur] = pltpu.async_copy(
            bufs[cur], out_hbm.at[wid * (ROWS_PER_W // SEQ) + bi,
                                  pl.ds(s0, CH), :], osem)
    o[0].wait()
    o[1].wait()


_gather_call = functools.partial(
    pl.kernel,
    out_type=[jax.ShapeDtypeStruct((BATCH, SEQ, EMB), jnp.float32)],
    mesh=plsc.VectorSubcoreMesh(core_axis_name="c", subcore_axis_name="s"),
    compiler_params=pltpu.CompilerParams(needs_layout_passes=False),
    scratch_types=[
        pltpu.VMEM((ROWS_PER_W,), jnp.int32),
        pltpu.VMEM((CH, EMB), jnp.float32),
        pltpu.VMEM((CH, EMB), jnp.float32),
        pltpu.VMEM_SHARED((SEQ, EMB), jnp.float32),
        pltpu.SemaphoreType.DMA,
        pltpu.SemaphoreType.DMA,
    ],
)(_gather_body)


_BB = 8


def _fea_body(x0_ref, x1_ref, wt_ref, o_ref):
    o_ref[:] = (x0_ref[:][:, :, None] * wt_ref[0][None, None, :]
                + x1_ref[:][:, :, None] * wt_ref[1][None, None, :])


def _fea_emb(x0, x1, wt):
    return pl.pallas_call(
        _fea_body,
        grid=(BATCH // _BB,),
        in_specs=[
            pl.BlockSpec((_BB, SEQ), lambda i: (i, 0)),
            pl.BlockSpec((_BB, SEQ), lambda i: (i, 0)),
            pl.BlockSpec((2, EMB), lambda i: (0, 0)),
        ],
        out_specs=pl.BlockSpec((_BB, SEQ, EMB), lambda i: (i, 0, 0)),
        out_shape=jax.ShapeDtypeStruct((BATCH, SEQ, EMB), jnp.float32),
    )(x0, x1, wt)


_FB = 32


def _fmt_body(vf_ref, t1_ref, t2_ref, vm_ref, tp_ref):
    vm_ref[:] = vf_ref[:].reshape(_FB, SEQ)
    t1r = t1_ref[:].reshape(_FB, 1, SEQ)
    t2r = t2_ref[:].reshape(_FB, 1, SEQ)
    tp_ref[:] = jnp.concatenate([t1r, t2r], axis=1)


def _fmt(vf, t1f, t2f):
    return pl.pallas_call(
        _fmt_body,
        grid=(BATCH // _FB,),
        in_specs=[
            pl.BlockSpec((_FB * SEQ,), lambda i: (i,)),
            pl.BlockSpec((_FB * SEQ,), lambda i: (i,)),
            pl.BlockSpec((_FB * SEQ,), lambda i: (i,)),
        ],
        out_specs=[
            pl.BlockSpec((_FB, SEQ), lambda i: (i, 0)),
            pl.BlockSpec((_FB, 2, SEQ), lambda i: (i, 0, 0)),
        ],
        out_shape=[
            jax.ShapeDtypeStruct((BATCH, SEQ), jnp.int32),
            jax.ShapeDtypeStruct((BATCH, 2, SEQ), jnp.int32),
        ],
    )(vf, t1f, t2f)


def kernel(x, solution, W, pattern, clac_stacks):
    zv = jnp.zeros((NLANE * SEQ,), jnp.int32)
    zl = jnp.zeros((NNODE * NLANE,), jnp.int32)
    visit_flat, t1f, t2f = _scan_call(solution.reshape(-1), zv, zl)

    (pos_emb,) = _gather_call(visit_flat, pattern)

    fea_emb = _fea_emb(x[:, :, 0], x[:, :, 1], W.T)
    visit_mod, tp = _fmt(visit_flat, t1f, t2f)
    top2_out = tp.transpose(0, 2, 1) * jnp.asarray(clac_stacks, jnp.int32)
    return (fea_emb, pos_emb, visit_mod, top2_out)
